# Initial kernel scaffold; baseline (speedup 1.0000x reference)
import jax
import jax.numpy as jnp
from jax.experimental import pallas as pl


def kernel(X, k, W, b):
    raise NotImplementedError("write your pallas kernel here")


# speedup vs baseline: 3.6090x; 3.6090x over previous
import jax
import jax.numpy as jnp
from jax.experimental import pallas as pl
from jax.experimental.pallas import tpu as pltpu

_GAMMA = 0.01618
_TOPK_CAP = 32


def _mm_kernel(x_ref, w_ref, b_ref, o_ref):
    y = jax.lax.dot_general(
        x_ref[...], w_ref[...],
        dimension_numbers=(((1,), (1,)), ((), ())),
        preferred_element_type=jnp.float32,
    )
    y = y + b_ref[...]
    o_ref[...] = jnp.maximum(y, 0.01 * y)


def _scan_kernel(k_ref, h_ref, o_ref):
    kk = jnp.minimum(k_ref[0], _TOPK_CAP)
    B, R, _ = h_ref.shape

    def body(bi, phi):
        h = h_ref[bi]
        s = h * phi
        sp = jnp.maximum(s, 0.0)
        keys = jax.lax.bitcast_convert_type(sp, jnp.int32)
        t = jnp.int32(0)
        for bit in range(30, -1, -1):
            tt = jnp.bitwise_or(t, jnp.int32(1 << bit))
            cnt = jnp.sum((keys >= tt).astype(jnp.int32))
            t = jnp.where(cnt >= kk, tt, t)
        mask = (keys >= t) & (s > 0.0) & (kk >= 1)
        binf = mask.astype(jnp.float32)
        o_ref[bi] = binf
        return jnp.minimum(phi + _GAMMA, 1.0) * (1.0 - binf)

    jax.lax.fori_loop(0, B, body, jnp.ones((R, 128), jnp.float32))


def kernel(X, k, W, b):
    B, D_in = X.shape
    D_out = W.shape[0]
    BN = 2048

    H = pl.pallas_call(
        _mm_kernel,
        grid=(D_out // BN,),
        in_specs=[
            pl.BlockSpec((B, D_in), lambda i: (0, 0)),
            pl.BlockSpec((BN, D_in), lambda i: (i, 0)),
            pl.BlockSpec((1, BN), lambda i: (0, i)),
        ],
        out_specs=pl.BlockSpec((B, BN), lambda i: (0, i)),
        out_shape=jax.ShapeDtypeStruct((B, D_out), jnp.float32),
    )(X, W, b.reshape(1, D_out))

    H3 = H.reshape(B, D_out // 128, 128)
    out3 = pl.pallas_call(
        _scan_kernel,
        in_specs=[
            pl.BlockSpec(memory_space=pltpu.SMEM),
            pl.BlockSpec(memory_space=pltpu.VMEM),
        ],
        out_specs=pl.BlockSpec(memory_space=pltpu.VMEM),
        out_shape=jax.ShapeDtypeStruct((B, D_out // 128, 128), jnp.float32),
    )(jnp.asarray(k, jnp.int32).reshape(1), H3)
    return out3.reshape(B, D_out)


# speedup vs baseline: 5.0539x; 1.4004x over previous
import jax
import jax.numpy as jnp
from jax.experimental import pallas as pl
from jax.experimental.pallas import tpu as pltpu

_GAMMA = 0.01618
_TOPK_CAP = 32


def _mm_kernel(x_ref, w_ref, b_ref, o_ref):
    y = jax.lax.dot_general(
        x_ref[...], w_ref[...],
        dimension_numbers=(((1,), (1,)), ((), ())),
        preferred_element_type=jnp.float32,
    )
    y = y + b_ref[...]
    o_ref[...] = jnp.maximum(y, 0.01 * y)


def _scan_kernel(k_ref, h_ref, o_ref):
    kk = jnp.minimum(k_ref[0], _TOPK_CAP)
    B, R, _ = h_ref.shape

    def body(bi, phi):
        h = h_ref[bi]
        s = h * phi
        sp = jnp.maximum(s, 0.0)
        keys = jax.lax.bitcast_convert_type(sp, jnp.int32)
        t = jnp.int32(0)
        for sh in range(29, -1, -2):
            cnts = [
                jnp.sum((keys >= (t + jnp.int32(c << sh))).astype(jnp.int32))
                for c in (1, 2, 3)
            ]
            d = sum((c >= kk).astype(jnp.int32) for c in cnts)
            t = t + (d << sh)
        tt = jnp.bitwise_or(t, jnp.int32(1))
        cnt = jnp.sum((keys >= tt).astype(jnp.int32))
        t = jnp.where(cnt >= kk, tt, t)
        mask = (keys >= t) & (s > 0.0) & (kk >= 1)
        binf = mask.astype(jnp.float32)
        o_ref[bi] = binf
        return jnp.minimum(phi + _GAMMA, 1.0) * (1.0 - binf)

    jax.lax.fori_loop(0, B, body, jnp.ones((R, 128), jnp.float32))


def kernel(X, k, W, b):
    B, D_in = X.shape
    D_out = W.shape[0]
    BN = 2048

    H = pl.pallas_call(
        _mm_kernel,
        grid=(D_out // BN,),
        in_specs=[
            pl.BlockSpec((B, D_in), lambda i: (0, 0)),
            pl.BlockSpec((BN, D_in), lambda i: (i, 0)),
            pl.BlockSpec((1, BN), lambda i: (0, i)),
        ],
        out_specs=pl.BlockSpec((B, BN), lambda i: (0, i)),
        out_shape=jax.ShapeDtypeStruct((B, D_out), jnp.float32),
    )(X, W, b.reshape(1, D_out))

    H3 = H.reshape(B, D_out // 128, 128)
    out3 = pl.pallas_call(
        _scan_kernel,
        in_specs=[
            pl.BlockSpec(memory_space=pltpu.SMEM),
            pl.BlockSpec(memory_space=pltpu.VMEM),
        ],
        out_specs=pl.BlockSpec(memory_space=pltpu.VMEM),
        out_shape=jax.ShapeDtypeStruct((B, D_out // 128, 128), jnp.float32),
    )(jnp.asarray(k, jnp.int32).reshape(1), H3)
    return out3.reshape(B, D_out)


# speedup vs baseline: 6.2290x; 1.2325x over previous
import jax
import jax.numpy as jnp
from jax.experimental import pallas as pl
from jax.experimental.pallas import tpu as pltpu

_GAMMA = 0.01618
_TOPK_CAP = 32


def _fused_kernel(k_ref, x_ref, w_ref, b_ref, o_ref, h3_ref):
    i = pl.program_id(0)
    nb = pl.num_programs(0) - 1
    B = x_ref.shape[0]
    R = h3_ref.shape[1]
    rb = w_ref.shape[0] // 128

    @pl.when(i < nb)
    def _mm():
        y = jax.lax.dot_general(
            x_ref[...], w_ref[...],
            dimension_numbers=(((1,), (1,)), ((), ())),
            preferred_element_type=jnp.float32,
        )
        y = y + b_ref[...]
        y = jnp.maximum(y, 0.01 * y)
        h3_ref[:, pl.ds(i * rb, rb), :] = y.reshape(B, rb, 128)

    @pl.when(i == nb)
    def _scan():
        kk = jnp.minimum(k_ref[0], _TOPK_CAP)

        def body(bi, phi):
            h = h3_ref[bi]
            s = h * phi
            sp = jnp.maximum(s, 0.0)
            keys = jax.lax.bitcast_convert_type(sp, jnp.int32)
            t = jnp.int32(0)
            for sh in range(28, 0, -3):
                cnts = [
                    jnp.sum(
                        (keys >= (t + jnp.int32(c << sh))).astype(jnp.int32))
                    for c in range(1, 8)
                ]
                d = sum((c >= kk).astype(jnp.int32) for c in cnts)
                t = t + (d << sh)
            tt = jnp.bitwise_or(t, jnp.int32(1))
            cnt = jnp.sum((keys >= tt).astype(jnp.int32))
            t = jnp.where(cnt >= kk, tt, t)
            mask = (keys >= t) & (s > 0.0) & (kk >= 1)
            binf = mask.astype(jnp.float32)
            o_ref[pl.ds(bi, 1), :] = binf.reshape(1, R * 128)
            return jnp.minimum(phi + _GAMMA, 1.0) * (1.0 - binf)

        jax.lax.fori_loop(0, B, body, jnp.ones((R, 128), jnp.float32))


def kernel(X, k, W, b):
    B, D_in = X.shape
    D_out = W.shape[0]
    BN = 2048
    nb = D_out // BN

    return pl.pallas_call(
        _fused_kernel,
        grid=(nb + 1,),
        in_specs=[
            pl.BlockSpec(memory_space=pltpu.SMEM),
            pl.BlockSpec((B, D_in), lambda i: (0, 0)),
            pl.BlockSpec((BN, D_in), lambda i: (jnp.minimum(i, nb - 1), 0)),
            pl.BlockSpec((1, BN), lambda i: (0, jnp.minimum(i, nb - 1))),
        ],
        out_specs=pl.BlockSpec((B, D_out), lambda i: (0, 0)),
        out_shape=jax.ShapeDtypeStruct((B, D_out), jnp.float32),
        scratch_shapes=[pltpu.VMEM((B, D_out // 128, 128), jnp.float32)],
    )(jnp.asarray(k, jnp.int32).reshape(1), X, W, b.reshape(1, D_out))


# speedup vs baseline: 8.9265x; 1.4331x over previous
import jax
import jax.numpy as jnp
from jax.experimental import pallas as pl
from jax.experimental.pallas import tpu as pltpu

_GAMMA = 0.01618
_TOPK_CAP = 32


def _fused_kernel(k_ref, x_ref, w_ref, b_ref, o_ref, h3_ref):
    i = pl.program_id(0)
    nb = pl.num_programs(0) - 1
    B = x_ref.shape[0]
    R = h3_ref.shape[1]
    rb = w_ref.shape[0] // 128

    @pl.when(i < nb)
    def _mm():
        y = jax.lax.dot_general(
            x_ref[...], w_ref[...],
            dimension_numbers=(((1,), (1,)), ((), ())),
            preferred_element_type=jnp.float32,
        )
        y = y + b_ref[...]
        y = jnp.maximum(y, 0.01 * y)
        h3_ref[:, pl.ds(i * rb, rb), :] = y.reshape(B, rb, 128)

    @pl.when(i == nb)
    def _scan():
        kk = jnp.minimum(k_ref[0], _TOPK_CAP)

        def body(bi, phi):
            h = h3_ref[bi]
            s = h * phi
            sp = jnp.maximum(s, 0.0)
            keys = jax.lax.bitcast_convert_type(sp, jnp.int32)
            def round_body(st):
                ri, t, found, tf = st
                sh = 28 - 3 * ri
                one = jnp.int32(1)
                cnts = [
                    jnp.sum((keys >= (t + jax.lax.shift_left(
                        jnp.int32(c), sh))).astype(jnp.int32))
                    for c in range(1, 8)
                ]
                d = jnp.int32(0)
                for c, cnt_c in enumerate(cnts, start=1):
                    tt_c = t + jax.lax.shift_left(jnp.int32(c), sh)
                    hit = cnt_c == kk
                    tf = jnp.where(hit & ~found, tt_c, tf)
                    found = found | hit
                    d = d + (cnt_c >= kk).astype(jnp.int32)
                return (ri + one, t + jax.lax.shift_left(d, sh), found, tf)

            st = (jnp.int32(0), jnp.int32(0), jnp.bool_(False), jnp.int32(0))
            ri, t, found, tf = jax.lax.while_loop(
                lambda st: (st[0] < 10) & ~st[2], round_body, st)
            tt = jnp.bitwise_or(t, jnp.int32(1))
            cnt = jnp.sum((keys >= tt).astype(jnp.int32))
            t = jnp.where(found, tf, jnp.where(cnt >= kk, tt, t))
            mask = (keys >= t) & (s > 0.0) & (kk >= 1)
            binf = mask.astype(jnp.float32)
            o_ref[pl.ds(bi, 1), :] = binf.reshape(1, R * 128)
            return jnp.minimum(phi + _GAMMA, 1.0) * (1.0 - binf)

        jax.lax.fori_loop(0, B, body, jnp.ones((R, 128), jnp.float32))


def kernel(X, k, W, b):
    B, D_in = X.shape
    D_out = W.shape[0]
    BN = 2048
    nb = D_out // BN

    return pl.pallas_call(
        _fused_kernel,
        grid=(nb + 1,),
        in_specs=[
            pl.BlockSpec(memory_space=pltpu.SMEM),
            pl.BlockSpec((B, D_in), lambda i: (0, 0)),
            pl.BlockSpec((BN, D_in), lambda i: (jnp.minimum(i, nb - 1), 0)),
            pl.BlockSpec((1, BN), lambda i: (0, jnp.minimum(i, nb - 1))),
        ],
        out_specs=pl.BlockSpec((B, D_out), lambda i: (0, 0)),
        out_shape=jax.ShapeDtypeStruct((B, D_out), jnp.float32),
        scratch_shapes=[pltpu.VMEM((B, D_out // 128, 128), jnp.float32)],
    )(jnp.asarray(k, jnp.int32).reshape(1), X, W, b.reshape(1, D_out))


# speedup vs baseline: 8.9622x; 1.0040x over previous
import jax
import jax.numpy as jnp
from jax.experimental import pallas as pl
from jax.experimental.pallas import tpu as pltpu

_GAMMA = 0.01618
_TOPK_CAP = 32


def _fused_kernel(k_ref, x_ref, w_ref, b_ref, o_ref, h3_ref):
    i = pl.program_id(0)
    nb = pl.num_programs(0) - 1
    B = x_ref.shape[0]
    R = h3_ref.shape[1]
    rb = w_ref.shape[0] // 128

    @pl.when(i < nb)
    def _mm():
        y = jax.lax.dot_general(
            x_ref[...], w_ref[...],
            dimension_numbers=(((1,), (1,)), ((), ())),
            preferred_element_type=jnp.float32,
        )
        y = y + b_ref[...]
        y = jnp.maximum(y, 0.01 * y)
        h3_ref[:, pl.ds(i * rb, rb), :] = y.reshape(B, rb, 128)

    @pl.when(i == nb)
    def _scan():
        kk = jnp.minimum(k_ref[0], _TOPK_CAP)

        def body(bi, phi):
            h = h3_ref[bi]
            s = h * phi
            keys = jax.lax.bitcast_convert_type(s, jnp.int32)
            def round_body(st):
                ri, t, found, tf = st
                sh = 28 - 3 * ri
                one = jnp.int32(1)
                cnts = [
                    jnp.sum((keys >= (t + jax.lax.shift_left(
                        jnp.int32(c), sh))).astype(jnp.int32))
                    for c in range(1, 8)
                ]
                d = jnp.int32(0)
                for c, cnt_c in enumerate(cnts, start=1):
                    tt_c = t + jax.lax.shift_left(jnp.int32(c), sh)
                    hit = cnt_c == kk
                    tf = jnp.where(hit & ~found, tt_c, tf)
                    found = found | hit
                    d = d + (cnt_c >= kk).astype(jnp.int32)
                return (ri + one, t + jax.lax.shift_left(d, sh), found, tf)

            st = (jnp.int32(0), jnp.int32(0), jnp.bool_(False), jnp.int32(0))
            ri, t, found, tf = jax.lax.while_loop(
                lambda st: (st[0] < 10) & ~st[2], round_body, st)
            tt = jnp.bitwise_or(t, jnp.int32(1))
            cnt = jnp.sum((keys >= tt).astype(jnp.int32))
            t = jnp.where(found, tf, jnp.where(cnt >= kk, tt, t))
            mask = (keys >= t) & (s > 0.0) & (kk >= 1)
            binf = mask.astype(jnp.float32)
            o_ref[pl.ds(bi, 1), :] = binf.reshape(1, R * 128)
            return jnp.where(mask, 0.0, jnp.minimum(phi + _GAMMA, 1.0))

        jax.lax.fori_loop(0, B, body, jnp.ones((R, 128), jnp.float32))


def kernel(X, k, W, b):
    B, D_in = X.shape
    D_out = W.shape[0]
    BN = 2048
    nb = D_out // BN

    return pl.pallas_call(
        _fused_kernel,
        grid=(nb + 1,),
        in_specs=[
            pl.BlockSpec(memory_space=pltpu.SMEM),
            pl.BlockSpec((B, D_in), lambda i: (0, 0)),
            pl.BlockSpec((BN, D_in), lambda i: (jnp.minimum(i, nb - 1), 0)),
            pl.BlockSpec((1, BN), lambda i: (0, jnp.minimum(i, nb - 1))),
        ],
        out_specs=pl.BlockSpec((B, D_out), lambda i: (0, 0)),
        out_shape=jax.ShapeDtypeStruct((B, D_out), jnp.float32),
        scratch_shapes=[pltpu.VMEM((B, D_out // 128, 128), jnp.float32)],
    )(jnp.asarray(k, jnp.int32).reshape(1), X, W, b.reshape(1, D_out))


# speedup vs baseline: 9.6289x; 1.0744x over previous
import jax
import jax.numpy as jnp
from jax.experimental import pallas as pl
from jax.experimental.pallas import tpu as pltpu

_GAMMA = 0.01618
_TOPK_CAP = 32


def _fused_kernel(k_ref, x_ref, w_ref, b_ref, o_ref, h3_ref):
    i = pl.program_id(0)
    nb = pl.num_programs(0) - 1
    B = x_ref.shape[0]
    R = h3_ref.shape[1]
    rb = w_ref.shape[0] // 128

    @pl.when(i < nb)
    def _mm():
        y = jax.lax.dot_general(
            x_ref[...], w_ref[...],
            dimension_numbers=(((1,), (1,)), ((), ())),
            preferred_element_type=jnp.float32,
        )
        y = y + b_ref[...]
        y = jnp.maximum(y, 0.01 * y)
        h3_ref[:, pl.ds(i * rb, rb), :] = y.reshape(B, rb, 128)

    @pl.when(i == nb)
    def _scan():
        kk = jnp.minimum(k_ref[0], _TOPK_CAP)

        def body(bi, carry):
            phi, prevt = carry
            h = h3_ref[bi]
            s = h * phi
            keys = jax.lax.bitcast_convert_type(s, jnp.int32)
            def round_body(st):
                ri, t, found, tf = st
                sh = 28 - 3 * ri
                one = jnp.int32(1)
                cnts = [
                    jnp.sum((keys >= (t + jax.lax.shift_left(
                        jnp.int32(c), sh))).astype(jnp.int32))
                    for c in range(1, 8)
                ]
                d = jnp.int32(0)
                for c, cnt_c in enumerate(cnts, start=1):
                    tt_c = t + jax.lax.shift_left(jnp.int32(c), sh)
                    hit = cnt_c == kk
                    tf = jnp.where(hit & ~found, tt_c, tf)
                    found = found | hit
                    d = d + (cnt_c >= kk).astype(jnp.int32)
                return (ri + one, t + jax.lax.shift_left(d, sh), found, tf)

            p0 = prevt & jnp.int32(~((1 << 22) - 1))
            c0 = jnp.sum((keys >= p0).astype(jnp.int32))
            c1 = jnp.sum((keys >= (p0 + (1 << 22))).astype(jnp.int32))
            warm = (p0 > 0) & (c0 >= kk) & (c1 < kk)
            found0 = warm & (c0 == kk)
            st = (jnp.where(warm, jnp.int32(2), jnp.int32(0)),
                  jnp.where(warm, p0, jnp.int32(0)),
                  found0,
                  jnp.where(found0, p0, jnp.int32(0)))
            ri, t, found, tf = jax.lax.while_loop(
                lambda st: (st[0] < 10) & ~st[2], round_body, st)
            tt = jnp.bitwise_or(t, jnp.int32(1))
            cnt = jnp.sum((keys >= tt).astype(jnp.int32))
            t = jnp.where(found, tf, jnp.where(cnt >= kk, tt, t))
            mask = (keys >= t) & (s > 0.0) & (kk >= 1)
            binf = mask.astype(jnp.float32)
            o_ref[pl.ds(bi, 1), :] = binf.reshape(1, R * 128)
            phi2 = jnp.where(mask, 0.0, jnp.minimum(phi + _GAMMA, 1.0))
            return (phi2, t)

        jax.lax.fori_loop(0, B, body,
                          (jnp.ones((R, 128), jnp.float32), jnp.int32(0)))


def kernel(X, k, W, b):
    B, D_in = X.shape
    D_out = W.shape[0]
    BN = 2048
    nb = D_out // BN

    return pl.pallas_call(
        _fused_kernel,
        grid=(nb + 1,),
        in_specs=[
            pl.BlockSpec(memory_space=pltpu.SMEM),
            pl.BlockSpec((B, D_in), lambda i: (0, 0)),
            pl.BlockSpec((BN, D_in), lambda i: (jnp.minimum(i, nb - 1), 0)),
            pl.BlockSpec((1, BN), lambda i: (0, jnp.minimum(i, nb - 1))),
        ],
        out_specs=pl.BlockSpec((B, D_out), lambda i: (0, 0)),
        out_shape=jax.ShapeDtypeStruct((B, D_out), jnp.float32),
        scratch_shapes=[pltpu.VMEM((B, D_out // 128, 128), jnp.float32)],
    )(jnp.asarray(k, jnp.int32).reshape(1), X, W, b.reshape(1, D_out))
